# Initial kernel scaffold; baseline (speedup 1.0000x reference)
import functools

import jax
import jax.numpy as jnp
import numpy as np
from jax import lax
from jax.experimental import pallas as pl
from jax.experimental.pallas import tpu as pltpu
from jax.experimental.pallas import tpu_sc as plsc

N = 50000
E = 800000
M0 = 8
M1 = 4
EA = 16
WN = 160

LANES = 128
E_PAD = 819200
N_ROWS = E_PAD // LANES
ROWS_PER_W = 200
GROUPS = 25
ROWS_PER_G = 8
EDGES_PER_G = ROWS_PER_G * LANES
NP = 50176
TILE_SLICE = NP // 16

TC_B = 1000
TC_GRID = E // TC_B

_INV_SQRT3 = float(1.0 / np.sqrt(3.0))
_INV_SQRT12 = float(1.0 / np.sqrt(12.0))
_INV_SQRT2 = float(1.0 / np.sqrt(2.0))

_PERM_IN = np.array(list(range(8)) + [8 + u * 3 + m for m in range(3) for u in range(4)], dtype=np.int32)
_PERM_OUT = np.array(list(range(8)) + [8 + m * 4 + u for u in range(4) for m in range(3)], dtype=np.int32)


def _sc_gather(node_p, src_rows):
    mesh = plsc.VectorSubcoreMesh(core_axis_name="c", subcore_axis_name="s")

    @functools.partial(
        pl.kernel,
        out_type=jax.ShapeDtypeStruct((E_PAD, 20), jnp.float32),
        mesh=mesh,
        scratch_types=[
            pltpu.VMEM((ROWS_PER_G, LANES), jnp.int32),
            pltpu.VMEM((EDGES_PER_G, 20), jnp.float32),
            pltpu.SemaphoreType.DMA,
        ],
    )
    def gather_k(node_hbm, src_hbm, out_hbm, idx_v, gbuf, sem):
        cid = lax.axis_index("c")
        sid = lax.axis_index("s")
        wid = cid * 16 + sid
        row0 = wid * ROWS_PER_W

        def body(g, carry):
            r = row0 + g * ROWS_PER_G
            pltpu.sync_copy(src_hbm.at[pl.ds(r, ROWS_PER_G)], idx_v)
            copies = []
            for j in range(ROWS_PER_G):
                copies.append(
                    pltpu.async_copy(
                        node_hbm.at[idx_v.at[j]],
                        gbuf.at[pl.ds(j * LANES, LANES)],
                        sem,
                    )
                )
            for c in copies:
                c.wait()
            pltpu.sync_copy(gbuf, out_hbm.at[pl.ds(r * LANES, EDGES_PER_G)])
            return carry

        lax.fori_loop(0, GROUPS, body, 0)

    return gather_k(node_p, src_rows)


def _sc_scatter(msg, dst_rows, zeros_hbm):
    mesh = plsc.VectorSubcoreMesh(core_axis_name="c", subcore_axis_name="s")

    @functools.partial(
        pl.kernel,
        out_type=jax.ShapeDtypeStruct((2, NP, 20), jnp.float32),
        mesh=mesh,
        scratch_types=[
            pltpu.VMEM((ROWS_PER_G, LANES), jnp.int32),
            pltpu.VMEM((EDGES_PER_G, 20), jnp.float32),
            pltpu.VMEM_SHARED((NP, 20), jnp.float32),
            pltpu.SemaphoreType.DMA,
        ],
    )
    def scatter_k(msg_hbm, dst_hbm, zero_hbm, out_hbm, idx_v, mbuf, acc, sem):
        cid = lax.axis_index("c")
        sid = lax.axis_index("s")
        wid = cid * 16 + sid
        row0 = wid * ROWS_PER_W

        sl = pl.ds(sid * TILE_SLICE, TILE_SLICE)
        pltpu.sync_copy(zero_hbm.at[sl], acc.at[sl])
        plsc.subcore_barrier()

        def body(g, carry):
            r = row0 + g * ROWS_PER_G
            pltpu.sync_copy(dst_hbm.at[pl.ds(r, ROWS_PER_G)], idx_v)
            pltpu.async_copy(
                msg_hbm.at[pl.ds(r * LANES, EDGES_PER_G)], mbuf, sem
            ).wait()
            for j in range(ROWS_PER_G):
                pltpu.sync_copy(
                    mbuf.at[pl.ds(j * LANES, LANES)],
                    acc.at[idx_v.at[j]],
                    add=True,
                )
            return carry

        lax.fori_loop(0, GROUPS, body, 0)
        plsc.subcore_barrier()
        pltpu.sync_copy(acc.at[sl], out_hbm.at[cid, sl])

    return scatter_k(msg, dst_rows, zeros_hbm)


def _tc_body(ea_ref, sh_ref, g_ref, w1_ref, b1_ref, w2_ref, b2_ref, out_ref):
    ea = ea_ref[...]
    h = jnp.dot(ea, w1_ref[...], preferred_element_type=jnp.float32) + b1_ref[...]
    h = h * jax.nn.sigmoid(h)
    w = jnp.dot(h, w2_ref[...], preferred_element_type=jnp.float32) + b2_ref[...]

    g = g_ref[...]
    sh = sh_ref[...]
    x0 = g[:, 0:8]
    x1m = [g[:, 8 + 4 * m: 12 + 4 * m] for m in range(3)]
    s0 = sh[:, 0:1]
    s1 = [sh[:, 1 + m: 2 + m] for m in range(3)]

    x0s0 = x0 * s0
    out0 = w[:, 0:8] * x0s0[:, 0:1]
    for u in range(1, 8):
        out0 = out0 + w[:, 8 * u: 8 * u + 8] * x0s0[:, u: u + 1]
    dot = (x1m[0] * s1[0] + x1m[1] * s1[1] + x1m[2] * s1[2]) * _INV_SQRT3
    for u in range(4):
        out0 = out0 + w[:, 112 + 8 * u: 120 + 8 * u] * dot[:, u: u + 1]
    out0 = out0 * _INV_SQRT12

    t2 = w[:, 64:68] * x0[:, 0:1]
    for u in range(1, 8):
        t2 = t2 + w[:, 64 + 4 * u: 68 + 4 * u] * x0[:, u: u + 1]

    o1 = []
    for m in range(3):
        mp1, mp2 = (m + 1) % 3, (m + 2) % 3
        cross = (x1m[mp1] * s1[mp2] - x1m[mp2] * s1[mp1]) * _INV_SQRT2
        t3 = w[:, 96:100] * x1m[m][:, 0:1]
        t5 = w[:, 144:148] * cross[:, 0:1]
        for u in range(1, 4):
            t3 = t3 + w[:, 96 + 4 * u: 100 + 4 * u] * x1m[m][:, u: u + 1]
            t5 = t5 + w[:, 144 + 4 * u: 148 + 4 * u] * cross[:, u: u + 1]
        o1.append((t2 * s1[m] + t3 * s0 + t5) * 0.25)

    out_ref[...] = jnp.concatenate([out0] + o1, axis=1)


def _tc_compute(edge_attr, edge_sh, gathered, W1, b1, W2, b2):
    return pl.pallas_call(
        _tc_body,
        grid=(TC_GRID,),
        in_specs=[
            pl.BlockSpec((TC_B, EA), lambda i: (i, 0)),
            pl.BlockSpec((TC_B, 4), lambda i: (i, 0)),
            pl.BlockSpec((TC_B, 20), lambda i: (i, 0)),
            pl.BlockSpec((EA, EA), lambda i: (0, 0)),
            pl.BlockSpec((1, EA), lambda i: (0, 0)),
            pl.BlockSpec((EA, WN), lambda i: (0, 0)),
            pl.BlockSpec((1, WN), lambda i: (0, 0)),
        ],
        out_specs=pl.BlockSpec((TC_B, 20), lambda i: (i, 0)),
        out_shape=jax.ShapeDtypeStruct((E_PAD, 20), jnp.float32),
    )(edge_attr, edge_sh, gathered, W1, b1, W2, b2)


def _tc_add(partials):
    def body(a_ref, o_ref):
        o_ref[...] = a_ref[0] + a_ref[1]

    return pl.pallas_call(
        body,
        grid=(NP // 3136,),
        in_specs=[pl.BlockSpec((2, 3136, 20), lambda i: (0, i, 0))],
        out_specs=pl.BlockSpec((3136, 20), lambda i: (i, 0)),
        out_shape=jax.ShapeDtypeStruct((NP, 20), jnp.float32),
    )(partials)


def kernel(node_attr, edge_index, edge_attr, edge_sh, W1, b1, W2, b2):
    node_p = node_attr[:, _PERM_IN]
    src = edge_index[0]
    dst = edge_index[1]
    pad = E_PAD - E
    src_rows = jnp.concatenate([src, jnp.zeros((pad,), jnp.int32)]).reshape(N_ROWS, LANES)
    dst_rows = jnp.concatenate([dst, jnp.full((pad,), N, jnp.int32)]).reshape(N_ROWS, LANES)

    gathered = _sc_gather(node_p, src_rows)
    msg = _tc_compute(
        edge_attr, edge_sh, gathered, W1, b1.reshape(1, EA), W2, b2.reshape(1, WN)
    )
    partials = _sc_scatter(msg, dst_rows, jnp.zeros((NP, 20), jnp.float32))
    out_mm = _tc_add(partials)
    return out_mm[:N, _PERM_OUT]


# speedup vs baseline: 1.3537x; 1.3537x over previous
import functools

import jax
import jax.numpy as jnp
import numpy as np
from jax import lax
from jax.experimental import pallas as pl
from jax.experimental.pallas import tpu as pltpu
from jax.experimental.pallas import tpu_sc as plsc

N = 50000
E = 800000
M0 = 8
M1 = 4
EA = 16
WN = 160

LANES = 128
N_ROWS = E // LANES
N_ROWS_PAD = 6256
ROWS_PER_G = 8
EDGES_PER_G = ROWS_PER_G * LANES
BASE_ROWS = N_ROWS // 32
EXTRA_W = N_ROWS - 32 * BASE_ROWS
NP = 50176
TILE_SLICE = NP // 16
E_STORE = N_ROWS_PAD * LANES
DW = 32

TC_B = 1000
TC_GRID = E // TC_B

_INV_SQRT3 = float(1.0 / np.sqrt(3.0))
_INV_SQRT12 = float(1.0 / np.sqrt(12.0))
_INV_SQRT2 = float(1.0 / np.sqrt(2.0))

_PERM_IN = np.array(list(range(8)) + [8 + u * 3 + m for m in range(3) for u in range(4)], dtype=np.int32)
_PERM_OUT = np.array(list(range(8)) + [8 + m * 4 + u for u in range(4) for m in range(3)], dtype=np.int32)


def _worker_range(wid):
    row0 = wid * BASE_ROWS + jnp.minimum(wid, EXTRA_W)
    nrows = BASE_ROWS + jnp.where(wid < EXTRA_W, 1, 0)
    return row0, nrows


def _sc_gather(node_p, src_rows):
    mesh = plsc.VectorSubcoreMesh(core_axis_name="c", subcore_axis_name="s")

    @functools.partial(
        pl.kernel,
        out_type=jax.ShapeDtypeStruct((E_STORE, DW), jnp.float32),
        mesh=mesh,
        scratch_types=[
            pltpu.VMEM((ROWS_PER_G, LANES), jnp.int32),
            pltpu.VMEM((EDGES_PER_G, DW), jnp.float32),
            pltpu.SemaphoreType.DMA,
        ],
        compiler_params=pltpu.CompilerParams(use_tc_tiling_on_sc=False),
    )
    def gather_k(node_hbm, src_hbm, out_hbm, idx_v, gbuf, sem):
        cid = lax.axis_index("c")
        sid = lax.axis_index("s")
        wid = cid * 16 + sid
        row0, _ = _worker_range(wid)

        def body(g, carry):
            r = row0 + g * ROWS_PER_G
            pltpu.sync_copy(src_hbm.at[pl.ds(r, ROWS_PER_G)], idx_v)
            copies = []
            for j in range(ROWS_PER_G):
                copies.append(
                    pltpu.async_copy(
                        node_hbm.at[idx_v.at[j]],
                        gbuf.at[pl.ds(j * LANES, LANES)],
                        sem,
                    )
                )
            for c in copies:
                c.wait()
            pltpu.sync_copy(gbuf, out_hbm.at[pl.ds(r * LANES, EDGES_PER_G)])
            return carry

        lax.fori_loop(0, 25, body, 0)

    return gather_k(node_p, src_rows)


def _sc_scatter(msg, dst_rows, zeros_hbm):
    mesh = plsc.VectorSubcoreMesh(core_axis_name="c", subcore_axis_name="s")

    RG = 4
    EG = RG * LANES

    @functools.partial(
        pl.kernel,
        out_type=jax.ShapeDtypeStruct((2, NP, DW), jnp.float32),
        mesh=mesh,
        scratch_types=[
            pltpu.VMEM((RG, LANES), jnp.int32),
            pltpu.VMEM((EG, DW), jnp.float32),
            pltpu.VMEM_SHARED((NP, DW), jnp.float32),
            pltpu.SemaphoreType.DMA,
        ],
        compiler_params=pltpu.CompilerParams(use_tc_tiling_on_sc=False),
    )
    def scatter_k(msg_hbm, dst_hbm, zero_hbm, out_hbm, idx_v, mbuf, acc, sem):
        cid = lax.axis_index("c")
        sid = lax.axis_index("s")
        wid = cid * 16 + sid
        row0, nrows = _worker_range(wid)

        sl = pl.ds(sid * TILE_SLICE, TILE_SLICE)
        pltpu.sync_copy(zero_hbm.at[sl], acc.at[sl])
        plsc.subcore_barrier()

        def stage(r):
            pltpu.sync_copy(dst_hbm.at[pl.ds(r, RG)], idx_v)
            pltpu.async_copy(
                msg_hbm.at[pl.ds(r * LANES, EG)], mbuf, sem
            ).wait()

        def body(g, carry):
            r = row0 + g * RG
            stage(r)
            for j in range(RG):
                pltpu.sync_copy(
                    mbuf.at[pl.ds(j * LANES, LANES)],
                    acc.at[idx_v.at[j]],
                    add=True,
                )
            return carry

        lax.fori_loop(0, 48, body, 0)
        r_tail = row0 + 48 * RG
        stage(r_tail)

        def tail(j, carry):
            pltpu.sync_copy(
                mbuf.at[pl.ds(j * LANES, LANES)],
                acc.at[idx_v.at[j]],
                add=True,
            )
            return carry

        lax.fori_loop(0, nrows - 48 * RG, tail, 0)
        plsc.subcore_barrier()
        pltpu.sync_copy(acc.at[sl], out_hbm.at[cid, sl])

    return scatter_k(msg, dst_rows, zeros_hbm)


def _tc_body(ea_ref, sh_ref, g_ref, w1_ref, b1_ref, w2_ref, b2_ref, out_ref):
    ea = ea_ref[...]
    h = jnp.dot(ea, w1_ref[...], preferred_element_type=jnp.float32) + b1_ref[...]
    h = h * jax.nn.sigmoid(h)
    w = jnp.dot(h, w2_ref[...], preferred_element_type=jnp.float32) + b2_ref[...]

    g = g_ref[...]
    sh = sh_ref[...]
    x0 = g[:, 0:8]
    x1m = [g[:, 8 + 4 * m: 12 + 4 * m] for m in range(3)]
    s0 = sh[:, 0:1]
    s1 = [sh[:, 1 + m: 2 + m] for m in range(3)]

    x0s0 = x0 * s0
    out0 = w[:, 0:8] * x0s0[:, 0:1]
    for u in range(1, 8):
        out0 = out0 + w[:, 8 * u: 8 * u + 8] * x0s0[:, u: u + 1]
    dot = (x1m[0] * s1[0] + x1m[1] * s1[1] + x1m[2] * s1[2]) * _INV_SQRT3
    for u in range(4):
        out0 = out0 + w[:, 112 + 8 * u: 120 + 8 * u] * dot[:, u: u + 1]
    out0 = out0 * _INV_SQRT12

    t2 = w[:, 64:68] * x0[:, 0:1]
    for u in range(1, 8):
        t2 = t2 + w[:, 64 + 4 * u: 68 + 4 * u] * x0[:, u: u + 1]

    o1 = []
    for m in range(3):
        mp1, mp2 = (m + 1) % 3, (m + 2) % 3
        cross = (x1m[mp1] * s1[mp2] - x1m[mp2] * s1[mp1]) * _INV_SQRT2
        t3 = w[:, 96:100] * x1m[m][:, 0:1]
        t5 = w[:, 144:148] * cross[:, 0:1]
        for u in range(1, 4):
            t3 = t3 + w[:, 96 + 4 * u: 100 + 4 * u] * x1m[m][:, u: u + 1]
            t5 = t5 + w[:, 144 + 4 * u: 148 + 4 * u] * cross[:, u: u + 1]
        o1.append((t2 * s1[m] + t3 * s0 + t5) * 0.25)

    zpad = out0[:, 0:4] * 0.0
    out_ref[...] = jnp.concatenate([out0] + o1 + [zpad, zpad, zpad], axis=1)


def _tc_compute(edge_attr, edge_sh, gathered, W1, b1, W2, b2):
    return pl.pallas_call(
        _tc_body,
        grid=(TC_GRID,),
        in_specs=[
            pl.BlockSpec((TC_B, EA), lambda i: (i, 0)),
            pl.BlockSpec((TC_B, 4), lambda i: (i, 0)),
            pl.BlockSpec((TC_B, DW), lambda i: (i, 0)),
            pl.BlockSpec((EA, EA), lambda i: (0, 0)),
            pl.BlockSpec((1, EA), lambda i: (0, 0)),
            pl.BlockSpec((EA, WN), lambda i: (0, 0)),
            pl.BlockSpec((1, WN), lambda i: (0, 0)),
        ],
        out_specs=pl.BlockSpec((TC_B, DW), lambda i: (i, 0)),
        out_shape=jax.ShapeDtypeStruct((E_STORE, DW), jnp.float32),
    )(edge_attr, edge_sh, gathered, W1, b1, W2, b2)


def _tc_add(partials):
    def body(a_ref, o_ref):
        o_ref[...] = a_ref[0] + a_ref[1]

    return pl.pallas_call(
        body,
        grid=(NP // 3136,),
        in_specs=[pl.BlockSpec((2, 3136, DW), lambda i: (0, i, 0))],
        out_specs=pl.BlockSpec((3136, DW), lambda i: (i, 0)),
        out_shape=jax.ShapeDtypeStruct((NP, DW), jnp.float32),
    )(partials)


def kernel(node_attr, edge_index, edge_attr, edge_sh, W1, b1, W2, b2):
    node_p = jnp.pad(node_attr[:, _PERM_IN], ((0, 0), (0, DW - 20)))
    src = edge_index[0]
    dst = edge_index[1]
    pad = N_ROWS_PAD * LANES - E
    spread = (jnp.arange(pad, dtype=jnp.int32) * 61) % N
    src_rows = jnp.concatenate([src, spread]).reshape(N_ROWS_PAD, LANES)
    dst_rows = jnp.concatenate([dst, spread]).reshape(N_ROWS_PAD, LANES)

    gathered = _sc_gather(node_p, src_rows)
    msg = _tc_compute(
        edge_attr, edge_sh, gathered, W1, b1.reshape(1, EA), W2, b2.reshape(1, WN)
    )
    partials = _sc_scatter(msg, dst_rows, jnp.zeros((NP, DW), jnp.float32))
    out_mm = _tc_add(partials)
    return out_mm[:N, _PERM_OUT]


# speedup vs baseline: 10.5178x; 7.7699x over previous
import functools

import jax
import jax.numpy as jnp
import numpy as np
from jax import lax
from jax.experimental import pallas as pl
from jax.experimental.pallas import tpu as pltpu
from jax.experimental.pallas import tpu_sc as plsc

N = 50000
E = 800000
M0 = 8
M1 = 4
EA = 16
WN = 160

LANES = 128
N_ROWS = E // LANES
N_ROWS_PAD = 6256
ROWS_PER_G = 8
EDGES_PER_G = ROWS_PER_G * LANES
BASE_ROWS = N_ROWS // 32
EXTRA_W = N_ROWS - 32 * BASE_ROWS
NP = 50176
TILE_SLICE = NP // 16
E_STORE = N_ROWS_PAD * LANES
DW = 32

TC_B = 3200
TC_GRID = E // TC_B

_INV_SQRT3 = float(1.0 / np.sqrt(3.0))
_INV_SQRT12 = float(1.0 / np.sqrt(12.0))
_INV_SQRT2 = float(1.0 / np.sqrt(2.0))

_PERM_IN = np.array(list(range(8)) + [8 + u * 3 + m for m in range(3) for u in range(4)], dtype=np.int32)
_PERM_OUT = np.array(list(range(8)) + [8 + m * 4 + u for u in range(4) for m in range(3)], dtype=np.int32)


def _worker_range(wid):
    row0 = wid * BASE_ROWS + jnp.minimum(wid, EXTRA_W)
    nrows = BASE_ROWS + jnp.where(wid < EXTRA_W, 1, 0)
    return row0, nrows


def _sc_gather(node_p, src_rows):
    mesh = plsc.VectorSubcoreMesh(core_axis_name="c", subcore_axis_name="s")

    @functools.partial(
        pl.kernel,
        out_type=jax.ShapeDtypeStruct((E_STORE, DW), jnp.float32),
        mesh=mesh,
        scratch_types=[
            pltpu.VMEM((ROWS_PER_G, LANES), jnp.int32),
            pltpu.VMEM((EDGES_PER_G, DW), jnp.float32),
            pltpu.SemaphoreType.DMA,
        ],
        compiler_params=pltpu.CompilerParams(use_tc_tiling_on_sc=False),
    )
    def gather_k(node_hbm, src_hbm, out_hbm, idx_v, gbuf, sem):
        cid = lax.axis_index("c")
        sid = lax.axis_index("s")
        wid = cid * 16 + sid
        row0, _ = _worker_range(wid)

        def body(g, carry):
            r = row0 + g * ROWS_PER_G
            pltpu.sync_copy(src_hbm.at[pl.ds(r, ROWS_PER_G)], idx_v)
            copies = []
            for j in range(ROWS_PER_G):
                copies.append(
                    pltpu.async_copy(
                        node_hbm.at[idx_v.at[j]],
                        gbuf.at[pl.ds(j * LANES, LANES)],
                        sem,
                    )
                )
            for c in copies:
                c.wait()
            pltpu.sync_copy(gbuf, out_hbm.at[pl.ds(r * LANES, EDGES_PER_G)])
            return carry

        lax.fori_loop(0, 25, body, 0)

    return gather_k(node_p, src_rows)


def _sc_scatter(msg, dst_rows, zeros_hbm):
    mesh = plsc.VectorSubcoreMesh(core_axis_name="c", subcore_axis_name="s")

    RG = 4
    EG = RG * LANES

    @functools.partial(
        pl.kernel,
        out_type=jax.ShapeDtypeStruct((2, NP, DW), jnp.float32),
        mesh=mesh,
        scratch_types=[
            pltpu.VMEM((RG, LANES), jnp.int32),
            pltpu.VMEM((EG, DW), jnp.float32),
            pltpu.VMEM_SHARED((NP, DW), jnp.float32),
            pltpu.SemaphoreType.DMA,
        ],
        compiler_params=pltpu.CompilerParams(use_tc_tiling_on_sc=False),
    )
    def scatter_k(msg_hbm, dst_hbm, zero_hbm, out_hbm, idx_v, mbuf, acc, sem):
        cid = lax.axis_index("c")
        sid = lax.axis_index("s")
        wid = cid * 16 + sid
        row0, nrows = _worker_range(wid)

        sl = pl.ds(sid * TILE_SLICE, TILE_SLICE)
        pltpu.sync_copy(zero_hbm.at[sl], acc.at[sl])
        plsc.subcore_barrier()

        def stage(r):
            pltpu.sync_copy(dst_hbm.at[pl.ds(r, RG)], idx_v)
            pltpu.async_copy(
                msg_hbm.at[pl.ds(r * LANES, EG)], mbuf, sem
            ).wait()

        def body(g, carry):
            r = row0 + g * RG
            stage(r)
            for j in range(RG):
                pltpu.sync_copy(
                    mbuf.at[pl.ds(j * LANES, LANES)],
                    acc.at[idx_v.at[j]],
                    add=True,
                )
            return carry

        lax.fori_loop(0, 48, body, 0)
        r_tail = row0 + 48 * RG
        stage(r_tail)

        def tail(j, carry):
            pltpu.sync_copy(
                mbuf.at[pl.ds(j * LANES, LANES)],
                acc.at[idx_v.at[j]],
                add=True,
            )
            return carry

        lax.fori_loop(0, nrows - 48 * RG, tail, 0)
        plsc.subcore_barrier()
        pltpu.sync_copy(acc.at[sl], out_hbm.at[cid, sl])

    return scatter_k(msg, dst_rows, zeros_hbm)


def _tc_body(ea_ref, sh_ref, g_ref, w1t_ref, b1_ref, w2t_ref, b2_ref, out_ref):
    ea = ea_ref[...]
    h = jnp.dot(w1t_ref[...], ea, preferred_element_type=jnp.float32) + b1_ref[...]
    h = h * jax.nn.sigmoid(h)
    w = jnp.dot(w2t_ref[...], h, preferred_element_type=jnp.float32) + b2_ref[...]

    gt = g_ref[...].T
    sh = sh_ref[...]
    x0 = gt[0:8]
    x1m = [gt[8 + 4 * m: 12 + 4 * m] for m in range(3)]
    s0 = sh[0:1]
    s1 = [sh[1 + m: 2 + m] for m in range(3)]

    x0s0 = x0 * s0
    out0 = w[0:8] * x0s0[0:1]
    for u in range(1, 8):
        out0 = out0 + w[8 * u: 8 * u + 8] * x0s0[u: u + 1]
    dot = (x1m[0] * s1[0] + x1m[1] * s1[1] + x1m[2] * s1[2]) * _INV_SQRT3
    for u in range(4):
        out0 = out0 + w[112 + 8 * u: 120 + 8 * u] * dot[u: u + 1]
    out0 = out0 * _INV_SQRT12

    t2 = w[64:68] * x0[0:1]
    for u in range(1, 8):
        t2 = t2 + w[64 + 4 * u: 68 + 4 * u] * x0[u: u + 1]

    o1 = []
    for m in range(3):
        mp1, mp2 = (m + 1) % 3, (m + 2) % 3
        cross = (x1m[mp1] * s1[mp2] - x1m[mp2] * s1[mp1]) * _INV_SQRT2
        t3 = w[96:100] * x1m[m][0:1]
        t5 = w[144:148] * cross[0:1]
        for u in range(1, 4):
            t3 = t3 + w[96 + 4 * u: 100 + 4 * u] * x1m[m][u: u + 1]
            t5 = t5 + w[144 + 4 * u: 148 + 4 * u] * cross[u: u + 1]
        o1.append((t2 * s1[m] + t3 * s0 + t5) * 0.25)

    zpad = out0[0:4] * 0.0
    msg_t = jnp.concatenate([out0] + o1 + [zpad, zpad, zpad], axis=0)
    out_ref[...] = msg_t.T


def _tc_compute(ea_t, sh_t, gathered, W1t, b1c, W2t, b2c):
    return pl.pallas_call(
        _tc_body,
        grid=(TC_GRID,),
        in_specs=[
            pl.BlockSpec((EA, TC_B), lambda i: (0, i)),
            pl.BlockSpec((4, TC_B), lambda i: (0, i)),
            pl.BlockSpec((TC_B, DW), lambda i: (i, 0)),
            pl.BlockSpec((EA, EA), lambda i: (0, 0)),
            pl.BlockSpec((EA, 1), lambda i: (0, 0)),
            pl.BlockSpec((WN, EA), lambda i: (0, 0)),
            pl.BlockSpec((WN, 1), lambda i: (0, 0)),
        ],
        out_specs=pl.BlockSpec((TC_B, DW), lambda i: (i, 0)),
        out_shape=jax.ShapeDtypeStruct((E_STORE, DW), jnp.float32),
    )(ea_t, sh_t, gathered, W1t, b1c, W2t, b2c)


def _tc_add(partials):
    def body(a_ref, o_ref):
        o_ref[...] = a_ref[0] + a_ref[1]

    return pl.pallas_call(
        body,
        grid=(NP // 3136,),
        in_specs=[pl.BlockSpec((2, 3136, DW), lambda i: (0, i, 0))],
        out_specs=pl.BlockSpec((3136, DW), lambda i: (i, 0)),
        out_shape=jax.ShapeDtypeStruct((NP, DW), jnp.float32),
    )(partials)


def kernel(node_attr, edge_index, edge_attr, edge_sh, W1, b1, W2, b2):
    node_p = jnp.pad(node_attr[:, _PERM_IN], ((0, 0), (0, DW - 20)))
    src = edge_index[0]
    dst = edge_index[1]
    pad = N_ROWS_PAD * LANES - E
    spread = (jnp.arange(pad, dtype=jnp.int32) * 61) % N
    src_rows = jnp.concatenate([src, spread]).reshape(N_ROWS_PAD, LANES)
    dst_rows = jnp.concatenate([dst, spread]).reshape(N_ROWS_PAD, LANES)

    gathered = _sc_gather(node_p, src_rows)
    msg = _tc_compute(
        edge_attr.T, edge_sh.T, gathered,
        W1.T, b1.reshape(EA, 1), W2.T, b2.reshape(WN, 1),
    )
    partials = _sc_scatter(msg, dst_rows, jnp.zeros((NP, DW), jnp.float32))
    out_mm = _tc_add(partials)
    return out_mm[:N, _PERM_OUT]
